# Initial kernel scaffold; baseline (speedup 1.0000x reference)
import jax
import jax.numpy as jnp
from jax.experimental import pallas as pl


def kernel(x, W_enc, b_enc, W_dec):
    raise NotImplementedError("write your pallas kernel here")


# speedup vs baseline: 21.4229x; 21.4229x over previous
import jax
import jax.numpy as jnp
from jax.experimental import pallas as pl
from jax.experimental.pallas import tpu as pltpu

_K = 128


def _enc_kernel(x_ref, w_ref, b_ref, out_ref):
    acc = jax.lax.dot_general(
        x_ref[:], w_ref[:], (((1,), (1,)), ((), ())),
        preferred_element_type=jnp.float32)
    out_ref[:] = acc + b_ref[:]


def _thresh_kernel(pre_ref, t_ref, u_ref):
    u_ref[:] = jnp.maximum(pre_ref[:], 0.0)
    rows = u_ref.shape[0]
    lo0 = jnp.zeros((rows, 1), jnp.int32)
    hi0 = jnp.full((rows, 1), 0x7F800000, jnp.int32)

    def body(_, carry):
        lo, hi = carry
        mid = lo + ((hi - lo) >> 1)
        midf = jax.lax.bitcast_convert_type(mid, jnp.float32)
        cnt = jnp.sum((u_ref[:] >= midf).astype(jnp.float32),
                      axis=1, keepdims=True)
        pred = cnt >= float(_K)
        lo = jnp.where(pred, mid, lo)
        hi = jnp.where(pred, hi, mid)
        return lo, hi

    lo, _ = jax.lax.fori_loop(0, 31, body, (lo0, hi0))
    t_ref[:] = jax.lax.bitcast_convert_type(lo, jnp.float32)


def _dec_kernel(pre_ref, t_ref, w_ref, sparse_ref, recon_ref):
    j = pl.program_id(1)
    u = jnp.maximum(pre_ref[:], 0.0)
    sparse = jnp.where(u >= t_ref[:], u, 0.0)
    sparse_ref[:] = sparse
    contrib = jax.lax.dot_general(
        sparse, w_ref[:], (((1,), (1,)), ((), ())),
        preferred_element_type=jnp.float32)

    @pl.when(j == 0)
    def _():
        recon_ref[:] = jnp.zeros_like(recon_ref)

    recon_ref[:] += contrib


def kernel(x, W_enc, b_enc, W_dec):
    n, d = x.shape
    dict_size = W_enc.shape[0]

    bm_a = min(1024, n)
    bn_a = min(512, dict_size)
    rb = min(128, n)
    rc = min(1024, n)
    bn_c = min(512, dict_size)

    b2 = b_enc.reshape(1, dict_size)

    pre = pl.pallas_call(
        _enc_kernel,
        grid=(n // bm_a, dict_size // bn_a),
        in_specs=[
            pl.BlockSpec((bm_a, d), lambda i, j: (i, 0)),
            pl.BlockSpec((bn_a, d), lambda i, j: (j, 0)),
            pl.BlockSpec((1, bn_a), lambda i, j: (0, j)),
        ],
        out_specs=pl.BlockSpec((bm_a, bn_a), lambda i, j: (i, j)),
        out_shape=jax.ShapeDtypeStruct((n, dict_size), jnp.float32),
    )(x, W_enc, b2)

    t = pl.pallas_call(
        _thresh_kernel,
        grid=(n // rb,),
        in_specs=[pl.BlockSpec((rb, dict_size), lambda i: (i, 0))],
        out_specs=pl.BlockSpec((rb, 1), lambda i: (i, 0)),
        out_shape=jax.ShapeDtypeStruct((n, 1), jnp.float32),
        scratch_shapes=[pltpu.VMEM((rb, dict_size), jnp.float32)],
    )(pre)

    sparse, recon = pl.pallas_call(
        _dec_kernel,
        grid=(n // rc, dict_size // bn_c),
        in_specs=[
            pl.BlockSpec((rc, bn_c), lambda i, j: (i, j)),
            pl.BlockSpec((rc, 1), lambda i, j: (i, 0)),
            pl.BlockSpec((d, bn_c), lambda i, j: (0, j)),
        ],
        out_specs=[
            pl.BlockSpec((rc, bn_c), lambda i, j: (i, j)),
            pl.BlockSpec((rc, d), lambda i, j: (i, 0)),
        ],
        out_shape=[
            jax.ShapeDtypeStruct((n, dict_size), jnp.float32),
            jax.ShapeDtypeStruct((n, d), jnp.float32),
        ],
        compiler_params=pltpu.CompilerParams(
            dimension_semantics=("arbitrary", "arbitrary")),
    )(pre, t, W_dec)

    return recon, sparse


# speedup vs baseline: 28.6471x; 1.3372x over previous
import jax
import jax.numpy as jnp
from jax.experimental import pallas as pl
from jax.experimental.pallas import tpu as pltpu

_K = 128


def _bc_i32(v):
    return jax.lax.bitcast_convert_type(v, jnp.int32)


def _bc_f32(v):
    return jax.lax.bitcast_convert_type(v, jnp.float32)


def _enc_kernel(x_ref, w_ref, b_ref, out_ref):
    acc = jax.lax.dot_general(
        x_ref[:], w_ref[:], (((1,), (1,)), ((), ())),
        preferred_element_type=jnp.float32)
    out_ref[:] = jnp.maximum(acc + b_ref[:], 0.0)


def _thresh_kernel(u_ref, t_ref):
    rows, cols = u_ref.shape
    m = u_ref[:]
    s = cols // 2
    while s >= 128:
        m = jnp.maximum(m[:, :s], m[:, s:])
        s //= 2
    lo0 = _bc_i32(jnp.min(m, axis=1, keepdims=True))
    hi0 = _bc_i32(jnp.max(m, axis=1, keepdims=True)) + 1
    found0 = jnp.zeros((rows, 1), jnp.int32)
    ts0 = jnp.zeros((rows, 1), jnp.int32)

    def cond(c):
        lo, hi, found, _ = c
        return jnp.max((hi - lo) * (1 - found)) > 1

    def body(c):
        lo, hi, found, ts = c
        mid = lo + ((hi - lo) >> 1)
        midf = _bc_f32(mid)
        cnt = jnp.sum((u_ref[:] >= midf).astype(jnp.float32),
                      axis=1, keepdims=True)
        exact = jnp.where(cnt == float(_K), 1, 0)
        ts = jnp.where(exact * (1 - found) == 1, mid, ts)
        found = jnp.maximum(found, exact)
        pred = cnt >= float(_K)
        lo = jnp.where(pred, mid, lo)
        hi = jnp.where(pred, hi, mid)
        return lo, hi, found, ts

    lo, _, found, ts = jax.lax.while_loop(cond, body, (lo0, hi0, found0, ts0))
    t_ref[:] = _bc_f32(jnp.where(found == 1, ts, lo))


def _dec_kernel(u_ref, t_ref, w_ref, sparse_ref, recon_ref):
    j = pl.program_id(1)
    u = u_ref[:]
    sparse = jnp.where(u >= t_ref[:], u, 0.0)
    sparse_ref[:] = sparse
    contrib = jax.lax.dot_general(
        sparse, w_ref[:], (((1,), (1,)), ((), ())),
        preferred_element_type=jnp.float32)

    @pl.when(j == 0)
    def _():
        recon_ref[:] = jnp.zeros_like(recon_ref)

    recon_ref[:] += contrib


def kernel(x, W_enc, b_enc, W_dec):
    n, d = x.shape
    dict_size = W_enc.shape[0]

    bm_a = min(2048, n)
    bn_a = min(512, dict_size)
    rb = min(128, n)
    rc = min(1024, n)
    bn_c = min(512, dict_size)

    b2 = b_enc.reshape(1, dict_size)

    u = pl.pallas_call(
        _enc_kernel,
        grid=(n // bm_a, dict_size // bn_a),
        in_specs=[
            pl.BlockSpec((bm_a, d), lambda i, j: (i, 0)),
            pl.BlockSpec((bn_a, d), lambda i, j: (j, 0)),
            pl.BlockSpec((1, bn_a), lambda i, j: (0, j)),
        ],
        out_specs=pl.BlockSpec((bm_a, bn_a), lambda i, j: (i, j)),
        out_shape=jax.ShapeDtypeStruct((n, dict_size), jnp.float32),
    )(x, W_enc, b2)

    t = pl.pallas_call(
        _thresh_kernel,
        grid=(n // rb,),
        in_specs=[pl.BlockSpec((rb, dict_size), lambda i: (i, 0))],
        out_specs=pl.BlockSpec((rb, 1), lambda i: (i, 0)),
        out_shape=jax.ShapeDtypeStruct((n, 1), jnp.float32),
    )(u)

    sparse, recon = pl.pallas_call(
        _dec_kernel,
        grid=(n // rc, dict_size // bn_c),
        in_specs=[
            pl.BlockSpec((rc, bn_c), lambda i, j: (i, j)),
            pl.BlockSpec((rc, 1), lambda i, j: (i, 0)),
            pl.BlockSpec((d, bn_c), lambda i, j: (0, j)),
        ],
        out_specs=[
            pl.BlockSpec((rc, bn_c), lambda i, j: (i, j)),
            pl.BlockSpec((rc, d), lambda i, j: (i, 0)),
        ],
        out_shape=[
            jax.ShapeDtypeStruct((n, dict_size), jnp.float32),
            jax.ShapeDtypeStruct((n, d), jnp.float32),
        ],
        compiler_params=pltpu.CompilerParams(
            dimension_semantics=("arbitrary", "arbitrary")),
    )(u, t, W_dec)

    return recon, sparse
